# baseline (speedup 1.0000x reference)
import functools

import jax
import jax.numpy as jnp
from jax import lax
from jax.experimental import pallas as pl
from jax.experimental.pallas import tpu as pltpu
from jax.experimental.pallas import tpu_sc as plsc

N_FEAT = 128
N_CLASSES = 100000
N_ROWS = 4096
SCALE = 30.0
R_TILE = 1024
C_TILE = 3072
N_RTILES = N_ROWS // R_TILE
N_CTILES = pl.cdiv(N_CLASSES, C_TILE)
N_CHUNKS = C_TILE // N_FEAT
K_SLOTS = 2
RAG_BASE = (N_CTILES - 1) * C_TILE

_NC = 2
_NS = 16
_NW = _NC * _NS
_B_PER_W = N_ROWS // _NW


def _sc_gather(lut_hbm, tgt_hbm, out_hbm, idx_v, rows_v, sem):
    wid = lax.axis_index("s") * _NC + lax.axis_index("c")
    base = wid * _B_PER_W
    pltpu.sync_copy(tgt_hbm.at[pl.ds(base, _B_PER_W)], idx_v)
    pltpu.async_copy(lut_hbm.at[idx_v], rows_v, sem).wait()
    pltpu.sync_copy(rows_v, out_hbm.at[pl.ds(base, _B_PER_W)])


def _gather_rows(lut, targets):
    mesh = plsc.VectorSubcoreMesh(core_axis_name="c", subcore_axis_name="s")
    return functools.partial(
        pl.kernel,
        mesh=mesh,
        out_type=jax.ShapeDtypeStruct((N_ROWS, N_FEAT), jnp.float32),
        scratch_types=[
            pltpu.VMEM((_B_PER_W,), jnp.int32),
            pltpu.VMEM((_B_PER_W, N_FEAT), jnp.float32),
            pltpu.SemaphoreType.DMA,
        ],
    )(_sc_gather)(lut, targets)


def _fused_kernel(x_ref, g_ref, lut_ref, logits_ref, loss_ref,
                  m_ref, s_ref, scr_ref, sem):
    i = pl.program_id(0)
    j = pl.program_id(1)
    n = i * (N_CTILES - 1) + j - 1
    slot = lax.rem(n, K_SLOTS)
    rows = pl.ds(i * R_TILE, R_TILE)

    x = x_ref[...] * SCALE
    w = lut_ref[...]

    def _accumulate(lg, first):
        chunks = [lg[:, k * N_FEAT:(k + 1) * N_FEAT] for k in range(N_CHUNKS)]
        if first:
            m_new = chunks[0]
            for c in chunks[1:]:
                m_new = jnp.maximum(m_new, c)
            acc = jnp.exp(chunks[0] - m_new)
            for c in chunks[1:]:
                acc = acc + jnp.exp(c - m_new)
        else:
            m_old = m_ref[...]
            m_new = m_old
            for c in chunks:
                m_new = jnp.maximum(m_new, c)
            acc = s_ref[...] * jnp.exp(m_old - m_new)
            for c in chunks:
                acc = acc + jnp.exp(c - m_new)
        s_ref[...] = acc
        m_ref[...] = m_new

    @pl.when(j == 0)
    def _ragged_tile():
        logits = jax.lax.dot_general(
            x, w, (((1,), (1,)), ((), ())),
            preferred_element_type=jnp.float32)
        cols = RAG_BASE + jax.lax.broadcasted_iota(
            jnp.int32, (R_TILE, C_TILE), 1)
        _accumulate(jnp.where(cols < N_CLASSES, logits, -jnp.inf), True)

    @pl.when(j > 0)
    def _full_tile():
        @pl.when(n >= K_SLOTS)
        def _wait_reuse():
            pltpu.make_async_copy(
                scr_ref.at[slot],
                logits_ref.at[rows, pl.ds(0, C_TILE)],
                sem.at[slot]).wait()

        scr_ref[slot] = jax.lax.dot_general(
            x, w, (((1,), (1,)), ((), ())),
            preferred_element_type=jnp.float32)
        pltpu.make_async_copy(
            scr_ref.at[slot],
            logits_ref.at[rows, pl.ds((j - 1) * C_TILE, C_TILE)],
            sem.at[slot]).start()
        _accumulate(scr_ref[slot], False)

    @pl.when(j == N_CTILES - 1)
    def _finalize():
        m_l = m_ref[...]
        m_row = jnp.max(m_l, axis=1, keepdims=True)
        s_row = jnp.sum(s_ref[...] * jnp.exp(m_l - m_row),
                        axis=1, keepdims=True)
        tgt = jnp.sum(x * g_ref[...], axis=1, keepdims=True)
        part = jnp.sum(m_row + jnp.log(s_row) - tgt) / N_ROWS

        @pl.when(i == 0)
        def _first():
            loss_ref[0, 0] = part

        @pl.when(i > 0)
        def _rest():
            loss_ref[0, 0] = loss_ref[0, 0] + part

        @pl.when(i == N_RTILES - 1)
        def _drain():
            for k in range(K_SLOTS):
                pltpu.make_async_copy(
                    scr_ref.at[k],
                    logits_ref.at[rows, pl.ds(0, C_TILE)],
                    sem.at[k]).wait()


def _fixup_kernel(logits_in_ref, x_ref, lut_ref, logits_ref):
    del logits_in_ref
    x = x_ref[...] * SCALE
    w = lut_ref[...]
    logits_ref[...] = jax.lax.dot_general(
        x, w, (((1,), (1,)), ((), ())),
        preferred_element_type=jnp.float32)


def kernel(inputs, targets, lut):
    g_rows = _gather_rows(lut, targets.astype(jnp.int32))
    logits_main, loss = pl.pallas_call(
        _fused_kernel,
        grid=(N_RTILES, N_CTILES),
        in_specs=[
            pl.BlockSpec((R_TILE, N_FEAT), lambda i, j: (i, 0)),
            pl.BlockSpec((R_TILE, N_FEAT), lambda i, j: (i, 0)),
            pl.BlockSpec((C_TILE, N_FEAT),
                         lambda i, j: ((j + N_CTILES - 1) % N_CTILES, 0)),
        ],
        out_specs=[
            pl.BlockSpec(memory_space=pl.ANY),
            pl.BlockSpec(memory_space=pltpu.SMEM),
        ],
        out_shape=[
            jax.ShapeDtypeStruct((N_ROWS, N_CLASSES), jnp.float32),
            jax.ShapeDtypeStruct((1, 1), jnp.float32),
        ],
        scratch_shapes=[
            pltpu.VMEM((R_TILE, N_FEAT), jnp.float32),
            pltpu.VMEM((R_TILE, N_FEAT), jnp.float32),
            pltpu.VMEM((K_SLOTS, R_TILE, C_TILE), jnp.float32),
            pltpu.SemaphoreType.DMA((K_SLOTS,)),
        ],
        compiler_params=pltpu.CompilerParams(
            dimension_semantics=("arbitrary", "arbitrary")),
    )(inputs, g_rows, lut)

    logits = pl.pallas_call(
        _fixup_kernel,
        grid=(N_RTILES,),
        in_specs=[
            pl.BlockSpec(memory_space=pl.ANY),
            pl.BlockSpec((R_TILE, N_FEAT), lambda i: (i, 0)),
            pl.BlockSpec((C_TILE, N_FEAT), lambda i: (N_CTILES - 1, 0)),
        ],
        out_specs=pl.BlockSpec((R_TILE, C_TILE), lambda i: (i, N_CTILES - 1)),
        out_shape=jax.ShapeDtypeStruct((N_ROWS, N_CLASSES), jnp.float32),
        input_output_aliases={0: 0},
        compiler_params=pltpu.CompilerParams(
            dimension_semantics=("arbitrary",)),
    )(logits_main, inputs, lut)
    return loss[0, 0], logits


# speedup vs baseline: 1.0037x; 1.0037x over previous
import functools

import jax
import jax.numpy as jnp
from jax import lax
from jax.experimental import pallas as pl
from jax.experimental.pallas import tpu as pltpu
from jax.experimental.pallas import tpu_sc as plsc

N_FEAT = 128
N_CLASSES = 100000
N_ROWS = 4096
SCALE = 30.0
R_TILE = 1024
C_TILE = 2048
N_RTILES = N_ROWS // R_TILE
N_CTILES = pl.cdiv(N_CLASSES, C_TILE)
N_CHUNKS = C_TILE // N_FEAT
K_SLOTS = 4
RAG_BASE = (N_CTILES - 1) * C_TILE

_NC = 2
_NS = 16
_NW = _NC * _NS
_B_PER_W = N_ROWS // _NW


def _sc_gather(lut_hbm, tgt_hbm, out_hbm, idx_v, rows_v, sem):
    wid = lax.axis_index("s") * _NC + lax.axis_index("c")
    base = wid * _B_PER_W
    pltpu.sync_copy(tgt_hbm.at[pl.ds(base, _B_PER_W)], idx_v)
    pltpu.async_copy(lut_hbm.at[idx_v], rows_v, sem).wait()
    pltpu.sync_copy(rows_v, out_hbm.at[pl.ds(base, _B_PER_W)])


def _gather_rows(lut, targets):
    mesh = plsc.VectorSubcoreMesh(core_axis_name="c", subcore_axis_name="s")
    return functools.partial(
        pl.kernel,
        mesh=mesh,
        out_type=jax.ShapeDtypeStruct((N_ROWS, N_FEAT), jnp.float32),
        scratch_types=[
            pltpu.VMEM((_B_PER_W,), jnp.int32),
            pltpu.VMEM((_B_PER_W, N_FEAT), jnp.float32),
            pltpu.SemaphoreType.DMA,
        ],
    )(_sc_gather)(lut, targets)


def _fused_kernel(x_ref, g_ref, lut_ref, logits_ref, loss_ref,
                  m_ref, s_ref, scr_ref, sem):
    i = pl.program_id(0)
    j = pl.program_id(1)
    n = i * (N_CTILES - 1) + j - 1
    slot = lax.rem(n, K_SLOTS)
    rows = pl.ds(i * R_TILE, R_TILE)

    x = x_ref[...] * SCALE
    w = lut_ref[...]

    def _accumulate(lg, first):
        chunks = [lg[:, k * N_FEAT:(k + 1) * N_FEAT] for k in range(N_CHUNKS)]
        if first:
            m_new = chunks[0]
            for c in chunks[1:]:
                m_new = jnp.maximum(m_new, c)
            acc = jnp.exp(chunks[0] - m_new)
            for c in chunks[1:]:
                acc = acc + jnp.exp(c - m_new)
        else:
            m_old = m_ref[...]
            m_new = m_old
            for c in chunks:
                m_new = jnp.maximum(m_new, c)
            acc = s_ref[...] * jnp.exp(m_old - m_new)
            for c in chunks:
                acc = acc + jnp.exp(c - m_new)
        s_ref[...] = acc
        m_ref[...] = m_new

    @pl.when(j == 0)
    def _ragged_tile():
        logits = jax.lax.dot_general(
            x, w, (((1,), (1,)), ((), ())),
            preferred_element_type=jnp.float32)
        cols = RAG_BASE + jax.lax.broadcasted_iota(
            jnp.int32, (R_TILE, C_TILE), 1)
        _accumulate(jnp.where(cols < N_CLASSES, logits, -jnp.inf), True)

    @pl.when(j > 0)
    def _full_tile():
        @pl.when(n >= K_SLOTS)
        def _wait_reuse():
            pltpu.make_async_copy(
                scr_ref.at[slot],
                logits_ref.at[rows, pl.ds(0, C_TILE)],
                sem.at[slot]).wait()

        scr_ref[slot] = jax.lax.dot_general(
            x, w, (((1,), (1,)), ((), ())),
            preferred_element_type=jnp.float32)
        pltpu.make_async_copy(
            scr_ref.at[slot],
            logits_ref.at[rows, pl.ds((j - 1) * C_TILE, C_TILE)],
            sem.at[slot]).start()
        _accumulate(scr_ref[slot], False)

    @pl.when(j == N_CTILES - 1)
    def _finalize():
        m_l = m_ref[...]
        m_row = jnp.max(m_l, axis=1, keepdims=True)
        s_row = jnp.sum(s_ref[...] * jnp.exp(m_l - m_row),
                        axis=1, keepdims=True)
        tgt = jnp.sum(x * g_ref[...], axis=1, keepdims=True)
        part = jnp.sum(m_row + jnp.log(s_row) - tgt) / N_ROWS

        @pl.when(i == 0)
        def _first():
            loss_ref[0, 0] = part

        @pl.when(i > 0)
        def _rest():
            loss_ref[0, 0] = loss_ref[0, 0] + part

        @pl.when(i == N_RTILES - 1)
        def _drain():
            for k in range(K_SLOTS):
                pltpu.make_async_copy(
                    scr_ref.at[k],
                    logits_ref.at[rows, pl.ds(0, C_TILE)],
                    sem.at[k]).wait()


def _fixup_kernel(logits_in_ref, x_ref, lut_ref, logits_ref):
    del logits_in_ref
    x = x_ref[...] * SCALE
    w = lut_ref[...]
    logits_ref[...] = jax.lax.dot_general(
        x, w, (((1,), (1,)), ((), ())),
        preferred_element_type=jnp.float32)


def kernel(inputs, targets, lut):
    g_rows = _gather_rows(lut, targets.astype(jnp.int32))
    logits_main, loss = pl.pallas_call(
        _fused_kernel,
        grid=(N_RTILES, N_CTILES),
        in_specs=[
            pl.BlockSpec((R_TILE, N_FEAT), lambda i, j: (i, 0)),
            pl.BlockSpec((R_TILE, N_FEAT), lambda i, j: (i, 0)),
            pl.BlockSpec((C_TILE, N_FEAT),
                         lambda i, j: ((j + N_CTILES - 1) % N_CTILES, 0)),
        ],
        out_specs=[
            pl.BlockSpec(memory_space=pl.ANY),
            pl.BlockSpec(memory_space=pltpu.SMEM),
        ],
        out_shape=[
            jax.ShapeDtypeStruct((N_ROWS, N_CLASSES), jnp.float32),
            jax.ShapeDtypeStruct((1, 1), jnp.float32),
        ],
        scratch_shapes=[
            pltpu.VMEM((R_TILE, N_FEAT), jnp.float32),
            pltpu.VMEM((R_TILE, N_FEAT), jnp.float32),
            pltpu.VMEM((K_SLOTS, R_TILE, C_TILE), jnp.float32),
            pltpu.SemaphoreType.DMA((K_SLOTS,)),
        ],
        compiler_params=pltpu.CompilerParams(
            dimension_semantics=("arbitrary", "arbitrary")),
    )(inputs, g_rows, lut)

    logits = pl.pallas_call(
        _fixup_kernel,
        grid=(N_RTILES,),
        in_specs=[
            pl.BlockSpec(memory_space=pl.ANY),
            pl.BlockSpec((R_TILE, N_FEAT), lambda i: (i, 0)),
            pl.BlockSpec((C_TILE, N_FEAT), lambda i: (N_CTILES - 1, 0)),
        ],
        out_specs=pl.BlockSpec((R_TILE, C_TILE), lambda i: (i, N_CTILES - 1)),
        out_shape=jax.ShapeDtypeStruct((N_ROWS, N_CLASSES), jnp.float32),
        input_output_aliases={0: 0},
        compiler_params=pltpu.CompilerParams(
            dimension_semantics=("arbitrary",)),
    )(logits_main, inputs, lut)
    return loss[0, 0], logits
